# baseline (speedup 1.0000x reference)
import jax
import jax.numpy as jnp
from jax import lax
from jax.experimental import pallas as pl
from jax.experimental.pallas import tpu as pltpu
from jax.experimental.pallas import tpu_sc as plsc

F = 32
GB = 128
NWORK = 32
NBUF = 4
CORE0_SHARE = 0.32
NSUB = 16


def _sc_aggregate(h2, src_p, ridx_p, *, nba, nbb, n_nodes, ncg, r_rel,
                  with_deg):
    rnp = -(-(r_rel * n_nodes + 1) // (NSUB * GB)) * (NSUB * GB)
    zr = rnp // NSUB
    nblk = src_p.shape[1]
    nch = ncg * 4

    out_type = [jax.ShapeDtypeStruct((2, ncg, rnp, GB), jnp.float32)]
    if with_deg:
        out_type.append(jax.ShapeDtypeStruct((2, rnp, GB), jnp.float32))

    mesh = plsc.VectorSubcoreMesh(core_axis_name="c", subcore_axis_name="s",
                                  num_cores=2, num_subcores=NSUB)

    def body(h2_hbm, src_hbm, ridx_hbm, acc_hbm, *rest):
        if with_deg:
            deg_hbm = rest[0]
            rest = rest[1:]
        acc_sh, src2d, ridx2d, gidx, gbuf, zbuf = rest[:6]
        gsem = rest[6:6 + NBUF]
        ssem = rest[6 + NBUF:6 + 2 * NBUF]
        cid = lax.axis_index("c")
        sid = lax.axis_index("s")
        wid = sid * 2 + cid
        nbw = jnp.where(cid == 0, nba, nbb)

        pltpu.sync_copy(src_hbm.at[wid], src2d)
        pltpu.sync_copy(ridx_hbm.at[wid], ridx2d)

        def fill(ref, rows, val):
            def go(i, _):
                for k in range(F // 16):
                    ref[i, pl.ds(k * 16, 16)] = jnp.full((16,), val,
                                                         jnp.float32)
                return 0
            lax.fori_loop(0, rows, go, 0)
        fill(zbuf, GB, 0.0)

        def zero_stripe():
            def zz(k, _):
                pltpu.sync_copy(zbuf, acc_sh.at[pl.ds(sid * zr + k * GB, GB)])
                return 0
            lax.fori_loop(0, zr // GB, zz, 0)

        my_rows = pl.ds(sid * zr, zr)

        if with_deg:
            fill(gbuf.at[0], GB, 1.0)
            zero_stripe()
            plsc.subcore_barrier()

            def deg_blk(j, _):
                @pl.when(j < nbw)
                def _():
                    pltpu.sync_copy(gbuf.at[0], acc_sh.at[ridx2d.at[j]],
                                    add=True)
                return 0
            lax.fori_loop(0, nblk, deg_blk, 0)
            plsc.subcore_barrier()
            pltpu.sync_copy(acc_sh.at[my_rows],
                            deg_hbm.at[cid, my_rows, pl.ds(0, F)])
            plsc.subcore_barrier()

        def chunk(c32, _):
            zero_stripe()
            base = (c32 >> 2) * (4 * n_nodes) + (c32 & 3)

            def gi(j, _):
                for k in range(GB // 16):
                    v = src2d[j, pl.ds(k * 16, 16)]
                    gidx[j, pl.ds(k * 16, 16)] = v * 4 + base
                return 0
            lax.fori_loop(0, nblk, gi, 0)
            plsc.subcore_barrier()

            for b in range(NBUF):
                @pl.when(b < nbw)
                def _(b=b):
                    pltpu.async_copy(h2_hbm.at[gidx.at[b]], gbuf.at[b],
                                     gsem[b])

            def ring(g, _):
                for b in range(NBUF):
                    j = g * NBUF + b

                    @pl.when(j < nbw)
                    def _(j=j, b=b):
                        pltpu.make_async_copy(h2_hbm.at[gidx.at[b]],
                                              gbuf.at[b], gsem[b]).wait()
                        pltpu.async_copy(gbuf.at[b], acc_sh.at[ridx2d.at[j]],
                                         ssem[b], add=True)
                        pltpu.make_async_copy(gbuf.at[b],
                                              acc_sh.at[ridx2d.at[j]],
                                              ssem[b]).wait()

                    @pl.when(j + NBUF < nbw)
                    def _(j=j, b=b):
                        pltpu.async_copy(h2_hbm.at[gidx.at[j + NBUF]],
                                         gbuf.at[b], gsem[b])
                return 0
            lax.fori_loop(0, (nblk + NBUF - 1) // NBUF, ring, 0)
            plsc.subcore_barrier()
            pltpu.sync_copy(
                acc_sh.at[my_rows],
                acc_hbm.at[cid, c32 >> 2, my_rows,
                           pl.ds((c32 & 3) * F, F)])
            plsc.subcore_barrier()
            return 0
        lax.fori_loop(0, nch, chunk, 0)

    fn = pl.kernel(
        body,
        out_type=tuple(out_type),
        mesh=mesh,
        scratch_types=(
            pltpu.VMEM_SHARED((rnp, F), jnp.float32),
            pltpu.VMEM((nblk, GB), jnp.int32),
            pltpu.VMEM((nblk, GB), jnp.int32),
            pltpu.VMEM((nblk, GB), jnp.int32),
            pltpu.VMEM((NBUF, GB, F), jnp.float32),
            pltpu.VMEM((GB, F), jnp.float32),
        ) + (pltpu.SemaphoreType.DMA,) * (2 * NBUF),
        compiler_params=pltpu.CompilerParams(use_tc_tiling_on_sc=False),
    )
    return fn(h2, src_p, ridx_p)


def _concat_kernel(x, node_type3, type_emb, *, bn):
    n, d = x.shape
    nt, td = type_emb.shape
    nb = n // bn
    ncg = (d + td) // 128

    def body(x_ref, nt_ref, te_ref, o_ref):
        ids = nt_ref[0]
        temb = jnp.zeros((bn, td), jnp.float32)
        for t in range(nt):
            temb = jnp.where(ids == t, te_ref[t][None, :], temb)
        for g in range(d // 128):
            o_ref[g] = x_ref[:, g * 128:(g + 1) * 128]
        o_ref[ncg - 1] = temb

    return pl.pallas_call(
        body,
        grid=(nb,),
        in_specs=[
            pl.BlockSpec((bn, d), lambda i: (i, 0)),
            pl.BlockSpec((1, bn, 1), lambda i: (i, 0, 0)),
            pl.BlockSpec((nt, td), lambda i: (0, 0)),
        ],
        out_specs=pl.BlockSpec((ncg, bn, 128), lambda i: (0, i, 0)),
        out_shape=jax.ShapeDtypeStruct((ncg, n, 128), jnp.float32),
    )(x, node_type3, type_emb)


def _repack_kernel(acc, deg, *, n_nodes, r_rel, bn):
    _, ncg, rnp, f = acc.shape
    rn = r_rel * n_nodes
    nb = rn // bn

    def body(a_ref, d_ref, o_ref):
        d = d_ref[0, :, 0:1] + d_ref[1, :, 0:1]
        recip = 1.0 / jnp.maximum(d, 1.0)
        o_ref[...] = (a_ref[0, 0] + a_ref[1, 0]) * recip

    return pl.pallas_call(
        body,
        grid=(ncg, nb),
        in_specs=[
            pl.BlockSpec((2, 1, bn, f), lambda c, i: (0, c, i, 0)),
            pl.BlockSpec((2, bn, f), lambda c, i: (0, i, 0)),
        ],
        out_specs=pl.BlockSpec((bn, f), lambda c, i: (i, c)),
        out_shape=jax.ShapeDtypeStruct((rn, ncg * f), jnp.float32),
    )(acc, deg)


def _layer_matmul(h3, aggs, w_self, w_rel, b, *, bn, out_cgm):
    ncg, n, _ = h3.shape
    d_in = ncg * 128
    r_rel = w_rel.shape[0]
    d_out = w_self.shape[1]
    nb = n // bn
    ocg = d_out // 128

    def body(h_ref, a_ref, ws_ref, wr_ref, b_ref, o_ref):
        h_blk = jnp.concatenate([h_ref[g] for g in range(ncg)], axis=1)
        out = jnp.dot(h_blk, ws_ref[...],
                      preferred_element_type=jnp.float32) + b_ref[0][None, :]
        for r in range(r_rel):
            out += jnp.dot(a_ref[r], wr_ref[r],
                           preferred_element_type=jnp.float32)
        out = jnp.maximum(out, 0.0)
        if out_cgm:
            for g in range(ocg):
                o_ref[g] = out[:, g * 128:(g + 1) * 128]
        else:
            o_ref[...] = out

    if out_cgm:
        out_spec = pl.BlockSpec((ocg, bn, 128), lambda i: (0, i, 0))
        out_shape = jax.ShapeDtypeStruct((ocg, n, 128), jnp.float32)
    else:
        out_spec = pl.BlockSpec((bn, d_out), lambda i: (i, 0))
        out_shape = jax.ShapeDtypeStruct((n, d_out), jnp.float32)

    return pl.pallas_call(
        body,
        grid=(nb,),
        in_specs=[
            pl.BlockSpec((ncg, bn, 128), lambda i: (0, i, 0)),
            pl.BlockSpec((r_rel, bn, d_in), lambda i: (0, i, 0)),
            pl.BlockSpec((d_in, d_out), lambda i: (0, 0)),
            pl.BlockSpec((r_rel, d_in, d_out), lambda i: (0, 0, 0)),
            pl.BlockSpec((1, d_out), lambda i: (0, 0)),
        ],
        out_specs=out_spec,
        out_shape=out_shape,
    )(h3, aggs, w_self, w_rel, b)


def kernel(x, edge_index, edge_type, node_type, type_emb,
           W_self0, W_rel0, b0, W_self1, W_rel1, b1):
    n, d = x.shape
    e = edge_index.shape[1]
    r_rel = W_rel0.shape[0]
    d_in0 = d + type_emb.shape[1]

    tb = -(-e // (GB * NWORK)) * NWORK
    pad = tb * GB - e
    nbt = tb // NSUB
    nba = max(1, min(nbt - 1, round(nbt * CORE0_SHARE)))
    nbb = nbt - nba
    nbmax = max(nba, nbb)
    src = jnp.concatenate([edge_index[0], jnp.zeros((pad,), jnp.int32)])
    ridx = edge_type * n + edge_index[1]
    ridx = jnp.concatenate([ridx, jnp.full((pad,), r_rel * n, jnp.int32)])

    def part(a):
        a = a.reshape(tb, GB)
        blk_a = a[:NSUB * nba].reshape(NSUB, nba, GB)
        blk_b = a[NSUB * nba:].reshape(NSUB, nbb, GB)
        blk_a = jnp.pad(blk_a, ((0, 0), (0, nbmax - nba), (0, 0)))
        blk_b = jnp.pad(blk_b, ((0, 0), (0, nbmax - nbb), (0, 0)))
        return jnp.stack([blk_a, blk_b], axis=1).reshape(NWORK, nbmax, GB)
    src_p = part(src)
    ridx_p = part(ridx)
    node_type3 = node_type.reshape(n // 400, 400, 1)

    h0 = _concat_kernel(x, node_type3, type_emb, bn=400)
    ncg0 = d_in0 // 128
    acc0, deg = _sc_aggregate(h0.reshape(ncg0 * n * 4, F), src_p, ridx_p,
                              nba=nba, nbb=nbb, n_nodes=n, ncg=ncg0,
                              r_rel=r_rel, with_deg=True)
    agg0 = _repack_kernel(acc0, deg, n_nodes=n, r_rel=r_rel, bn=2000)
    h1 = _layer_matmul(h0, agg0.reshape(r_rel, n, d_in0),
                       W_self0, W_rel0, b0.reshape(1, -1), bn=400,
                       out_cgm=True)

    ncg1 = h1.shape[0]
    (acc1,) = _sc_aggregate(h1.reshape(ncg1 * n * 4, F), src_p, ridx_p,
                            nba=nba, nbb=nbb, n_nodes=n, ncg=ncg1,
                            r_rel=r_rel, with_deg=False)
    agg1 = _repack_kernel(acc1, deg, n_nodes=n, r_rel=r_rel, bn=2000)
    return _layer_matmul(h1, agg1.reshape(r_rel, n, ncg1 * 128),
                         W_self1, W_rel1, b1.reshape(1, -1), bn=400,
                         out_cgm=False)


# speedup vs baseline: 1.0699x; 1.0699x over previous
import jax
import jax.numpy as jnp
from jax import lax
from jax.experimental import pallas as pl
from jax.experimental.pallas import tpu as pltpu
from jax.experimental.pallas import tpu_sc as plsc

F = 32
GB = 128
NWORK = 32
NBUF = 4
CORE0_SHARE = 0.68
NSUB = 16


def _sc_aggregate(h2, src_p, ridx_p, *, nba, nbb, n_nodes, ncg, r_rel,
                  with_deg):
    rnp = -(-(r_rel * n_nodes + 1) // (NSUB * GB)) * (NSUB * GB)
    zr = rnp // NSUB
    nblk = src_p.shape[1]
    nch = ncg * 4

    out_type = [jax.ShapeDtypeStruct((2, ncg, rnp, GB), jnp.float32)]
    if with_deg:
        out_type.append(jax.ShapeDtypeStruct((2, rnp, GB), jnp.float32))

    mesh = plsc.VectorSubcoreMesh(core_axis_name="c", subcore_axis_name="s",
                                  num_cores=2, num_subcores=NSUB)

    def body(h2_hbm, src_hbm, ridx_hbm, acc_hbm, *rest):
        if with_deg:
            deg_hbm = rest[0]
            rest = rest[1:]
        acc_sh, src2d, ridx2d, gidx, gbuf, zbuf = rest[:6]
        gsem = rest[6:6 + NBUF]
        ssem = rest[6 + NBUF:6 + 2 * NBUF]
        cid = lax.axis_index("c")
        sid = lax.axis_index("s")
        wid = sid * 2 + cid
        nbw = jnp.where(cid == 0, nba, nbb)

        pltpu.sync_copy(src_hbm.at[wid], src2d)
        pltpu.sync_copy(ridx_hbm.at[wid], ridx2d)

        def fill(ref, rows, val):
            def go(i, _):
                for k in range(F // 16):
                    ref[i, pl.ds(k * 16, 16)] = jnp.full((16,), val,
                                                         jnp.float32)
                return 0
            lax.fori_loop(0, rows, go, 0)
        fill(zbuf, GB, 0.0)

        def zero_stripe():
            def zz(k, _):
                pltpu.sync_copy(zbuf, acc_sh.at[pl.ds(sid * zr + k * GB, GB)])
                return 0
            lax.fori_loop(0, zr // GB, zz, 0)

        my_rows = pl.ds(sid * zr, zr)

        if with_deg:
            fill(gbuf.at[0], GB, 1.0)
            zero_stripe()
            plsc.subcore_barrier()

            def deg_blk(j, _):
                @pl.when(j < nbw)
                def _():
                    pltpu.sync_copy(gbuf.at[0], acc_sh.at[ridx2d.at[j]],
                                    add=True)
                return 0
            lax.fori_loop(0, nblk, deg_blk, 0)
            plsc.subcore_barrier()
            pltpu.sync_copy(acc_sh.at[my_rows],
                            deg_hbm.at[cid, my_rows, pl.ds(0, F)])
            plsc.subcore_barrier()

        def chunk(c32, _):
            zero_stripe()
            base = (c32 >> 2) * (4 * n_nodes) + (c32 & 3)

            def gi(j, _):
                for k in range(GB // 16):
                    v = src2d[j, pl.ds(k * 16, 16)]
                    gidx[j, pl.ds(k * 16, 16)] = v * 4 + base
                return 0
            lax.fori_loop(0, nblk, gi, 0)
            plsc.subcore_barrier()

            for b in range(NBUF):
                @pl.when(b < nbw)
                def _(b=b):
                    pltpu.async_copy(h2_hbm.at[gidx.at[b]], gbuf.at[b],
                                     gsem[b])

            def ring(g, _):
                for b in range(NBUF):
                    j = g * NBUF + b

                    @pl.when(j < nbw)
                    def _(j=j, b=b):
                        pltpu.make_async_copy(h2_hbm.at[gidx.at[b]],
                                              gbuf.at[b], gsem[b]).wait()
                        pltpu.async_copy(gbuf.at[b], acc_sh.at[ridx2d.at[j]],
                                         ssem[b], add=True)
                        pltpu.make_async_copy(gbuf.at[b],
                                              acc_sh.at[ridx2d.at[j]],
                                              ssem[b]).wait()

                    @pl.when(j + NBUF < nbw)
                    def _(j=j, b=b):
                        pltpu.async_copy(h2_hbm.at[gidx.at[j + NBUF]],
                                         gbuf.at[b], gsem[b])
                return 0
            lax.fori_loop(0, (nblk + NBUF - 1) // NBUF, ring, 0)
            plsc.subcore_barrier()
            pltpu.sync_copy(
                acc_sh.at[my_rows],
                acc_hbm.at[cid, c32 >> 2, my_rows,
                           pl.ds((c32 & 3) * F, F)])
            plsc.subcore_barrier()
            return 0
        lax.fori_loop(0, nch, chunk, 0)

    fn = pl.kernel(
        body,
        out_type=tuple(out_type),
        mesh=mesh,
        scratch_types=(
            pltpu.VMEM_SHARED((rnp, F), jnp.float32),
            pltpu.VMEM((nblk, GB), jnp.int32),
            pltpu.VMEM((nblk, GB), jnp.int32),
            pltpu.VMEM((nblk, GB), jnp.int32),
            pltpu.VMEM((NBUF, GB, F), jnp.float32),
            pltpu.VMEM((GB, F), jnp.float32),
        ) + (pltpu.SemaphoreType.DMA,) * (2 * NBUF),
        compiler_params=pltpu.CompilerParams(use_tc_tiling_on_sc=False),
    )
    return fn(h2, src_p, ridx_p)


def _concat_kernel(x, node_type3, type_emb, *, bn):
    n, d = x.shape
    nt, td = type_emb.shape
    nb = n // bn
    ncg = (d + td) // 128

    def body(x_ref, nt_ref, te_ref, o_ref):
        ids = nt_ref[0]
        temb = jnp.zeros((bn, td), jnp.float32)
        for t in range(nt):
            temb = jnp.where(ids == t, te_ref[t][None, :], temb)
        for g in range(d // 128):
            o_ref[g] = x_ref[:, g * 128:(g + 1) * 128]
        o_ref[ncg - 1] = temb

    return pl.pallas_call(
        body,
        grid=(nb,),
        in_specs=[
            pl.BlockSpec((bn, d), lambda i: (i, 0)),
            pl.BlockSpec((1, bn, 1), lambda i: (i, 0, 0)),
            pl.BlockSpec((nt, td), lambda i: (0, 0)),
        ],
        out_specs=pl.BlockSpec((ncg, bn, 128), lambda i: (0, i, 0)),
        out_shape=jax.ShapeDtypeStruct((ncg, n, 128), jnp.float32),
    )(x, node_type3, type_emb)


def _repack_kernel(acc, deg, *, n_nodes, r_rel, bn):
    _, ncg, rnp, f = acc.shape
    rn = r_rel * n_nodes
    nb = rn // bn

    def body(a_ref, d_ref, o_ref):
        d = d_ref[0, :, 0:1] + d_ref[1, :, 0:1]
        recip = 1.0 / jnp.maximum(d, 1.0)
        o_ref[...] = (a_ref[0, 0] + a_ref[1, 0]) * recip

    return pl.pallas_call(
        body,
        grid=(ncg, nb),
        in_specs=[
            pl.BlockSpec((2, 1, bn, f), lambda c, i: (0, c, i, 0)),
            pl.BlockSpec((2, bn, f), lambda c, i: (0, i, 0)),
        ],
        out_specs=pl.BlockSpec((bn, f), lambda c, i: (i, c)),
        out_shape=jax.ShapeDtypeStruct((rn, ncg * f), jnp.float32),
    )(acc, deg)


def _layer_matmul(h3, aggs, w_self, w_rel, b, *, bn, out_cgm):
    ncg, n, _ = h3.shape
    d_in = ncg * 128
    r_rel = w_rel.shape[0]
    d_out = w_self.shape[1]
    nb = n // bn
    ocg = d_out // 128

    def body(h_ref, a_ref, ws_ref, wr_ref, b_ref, o_ref):
        h_blk = jnp.concatenate([h_ref[g] for g in range(ncg)], axis=1)
        out = jnp.dot(h_blk, ws_ref[...],
                      preferred_element_type=jnp.float32) + b_ref[0][None, :]
        for r in range(r_rel):
            out += jnp.dot(a_ref[r], wr_ref[r],
                           preferred_element_type=jnp.float32)
        out = jnp.maximum(out, 0.0)
        if out_cgm:
            for g in range(ocg):
                o_ref[g] = out[:, g * 128:(g + 1) * 128]
        else:
            o_ref[...] = out

    if out_cgm:
        out_spec = pl.BlockSpec((ocg, bn, 128), lambda i: (0, i, 0))
        out_shape = jax.ShapeDtypeStruct((ocg, n, 128), jnp.float32)
    else:
        out_spec = pl.BlockSpec((bn, d_out), lambda i: (i, 0))
        out_shape = jax.ShapeDtypeStruct((n, d_out), jnp.float32)

    return pl.pallas_call(
        body,
        grid=(nb,),
        in_specs=[
            pl.BlockSpec((ncg, bn, 128), lambda i: (0, i, 0)),
            pl.BlockSpec((r_rel, bn, d_in), lambda i: (0, i, 0)),
            pl.BlockSpec((d_in, d_out), lambda i: (0, 0)),
            pl.BlockSpec((r_rel, d_in, d_out), lambda i: (0, 0, 0)),
            pl.BlockSpec((1, d_out), lambda i: (0, 0)),
        ],
        out_specs=out_spec,
        out_shape=out_shape,
    )(h3, aggs, w_self, w_rel, b)


def kernel(x, edge_index, edge_type, node_type, type_emb,
           W_self0, W_rel0, b0, W_self1, W_rel1, b1):
    n, d = x.shape
    e = edge_index.shape[1]
    r_rel = W_rel0.shape[0]
    d_in0 = d + type_emb.shape[1]

    tb = -(-e // (GB * NWORK)) * NWORK
    pad = tb * GB - e
    nbt = tb // NSUB
    nba = max(1, min(nbt - 1, round(nbt * CORE0_SHARE)))
    nbb = nbt - nba
    nbmax = max(nba, nbb)
    src = jnp.concatenate([edge_index[0], jnp.zeros((pad,), jnp.int32)])
    ridx = edge_type * n + edge_index[1]
    ridx = jnp.concatenate([ridx, jnp.full((pad,), r_rel * n, jnp.int32)])

    def part(a):
        a = a.reshape(tb, GB)
        blk_a = a[:NSUB * nba].reshape(NSUB, nba, GB)
        blk_b = a[NSUB * nba:].reshape(NSUB, nbb, GB)
        blk_a = jnp.pad(blk_a, ((0, 0), (0, nbmax - nba), (0, 0)))
        blk_b = jnp.pad(blk_b, ((0, 0), (0, nbmax - nbb), (0, 0)))
        return jnp.stack([blk_a, blk_b], axis=1).reshape(NWORK, nbmax, GB)
    src_p = part(src)
    ridx_p = part(ridx)
    node_type3 = node_type.reshape(n // 400, 400, 1)

    h0 = _concat_kernel(x, node_type3, type_emb, bn=400)
    ncg0 = d_in0 // 128
    acc0, deg = _sc_aggregate(h0.reshape(ncg0 * n * 4, F), src_p, ridx_p,
                              nba=nba, nbb=nbb, n_nodes=n, ncg=ncg0,
                              r_rel=r_rel, with_deg=True)
    agg0 = _repack_kernel(acc0, deg, n_nodes=n, r_rel=r_rel, bn=2000)
    h1 = _layer_matmul(h0, agg0.reshape(r_rel, n, d_in0),
                       W_self0, W_rel0, b0.reshape(1, -1), bn=400,
                       out_cgm=True)

    ncg1 = h1.shape[0]
    (acc1,) = _sc_aggregate(h1.reshape(ncg1 * n * 4, F), src_p, ridx_p,
                            nba=nba, nbb=nbb, n_nodes=n, ncg=ncg1,
                            r_rel=r_rel, with_deg=False)
    agg1 = _repack_kernel(acc1, deg, n_nodes=n, r_rel=r_rel, bn=2000)
    return _layer_matmul(h1, agg1.reshape(r_rel, n, ncg1 * 128),
                         W_self1, W_rel1, b1.reshape(1, -1), bn=400,
                         out_cgm=False)


# speedup vs baseline: 1.0890x; 1.0179x over previous
import jax
import jax.numpy as jnp
from jax import lax
from jax.experimental import pallas as pl
from jax.experimental.pallas import tpu as pltpu
from jax.experimental.pallas import tpu_sc as plsc

F = 32
GB = 128
NWORK = 32
NBUF = 4
CORE0_SHARE = 0.5
NSUB = 16


def _sc_aggregate(h2, src_p, ridx_p, *, nba, nbb, n_nodes, ncg, r_rel,
                  with_deg):
    rnp = -(-(r_rel * n_nodes + 1) // (NSUB * GB)) * (NSUB * GB)
    zr = rnp // NSUB
    nblk = src_p.shape[1]
    nch = ncg * 4

    out_type = [jax.ShapeDtypeStruct((2, ncg, rnp, GB), jnp.float32)]
    if with_deg:
        out_type.append(jax.ShapeDtypeStruct((2, rnp, GB), jnp.float32))

    mesh = plsc.VectorSubcoreMesh(core_axis_name="c", subcore_axis_name="s",
                                  num_cores=2, num_subcores=NSUB)

    def body(h2_hbm, src_hbm, ridx_hbm, acc_hbm, *rest):
        if with_deg:
            deg_hbm = rest[0]
            rest = rest[1:]
        acc_sh, src2d, ridx2d, gidx, gbuf, zbuf = rest[:6]
        gsem = rest[6:6 + NBUF]
        ssem = rest[6 + NBUF:6 + 2 * NBUF]
        cid = lax.axis_index("c")
        sid = lax.axis_index("s")
        wid = sid * 2 + cid
        nbw = jnp.where(cid == 0, nba, nbb)

        pltpu.sync_copy(src_hbm.at[wid], src2d)
        pltpu.sync_copy(ridx_hbm.at[wid], ridx2d)

        def fill(ref, rows, val):
            def go(i, _):
                for k in range(F // 16):
                    ref[i, pl.ds(k * 16, 16)] = jnp.full((16,), val,
                                                         jnp.float32)
                return 0
            lax.fori_loop(0, rows, go, 0)
        fill(zbuf, GB, 0.0)

        def zero_stripe():
            def zz(k, _):
                pltpu.sync_copy(zbuf, acc_sh.at[pl.ds(sid * zr + k * GB, GB)])
                return 0
            lax.fori_loop(0, zr // GB, zz, 0)

        my_rows = pl.ds(sid * zr, zr)

        if with_deg:
            fill(gbuf.at[0], GB, 1.0)
            zero_stripe()
            plsc.subcore_barrier()

            def deg_blk(j, _):
                @pl.when(j < nbw)
                def _():
                    pltpu.sync_copy(gbuf.at[0], acc_sh.at[ridx2d.at[j]],
                                    add=True)
                return 0
            lax.fori_loop(0, nblk, deg_blk, 0)
            plsc.subcore_barrier()
            pltpu.sync_copy(acc_sh.at[my_rows],
                            deg_hbm.at[cid, my_rows, pl.ds(0, F)])
            plsc.subcore_barrier()

        def chunk(c32, _):
            zero_stripe()
            base = (c32 >> 2) * (4 * n_nodes) + (c32 & 3)

            def gi(j, _):
                for k in range(GB // 16):
                    v = src2d[j, pl.ds(k * 16, 16)]
                    gidx[j, pl.ds(k * 16, 16)] = v * 4 + base
                return 0
            lax.fori_loop(0, nblk, gi, 0)
            plsc.subcore_barrier()

            for b in range(NBUF):
                @pl.when(b < nbw)
                def _(b=b):
                    pltpu.async_copy(h2_hbm.at[gidx.at[b]], gbuf.at[b],
                                     gsem[b])

            def ring(g, _):
                for b in range(NBUF):
                    j = g * NBUF + b

                    @pl.when(j < nbw)
                    def _(j=j, b=b):
                        pltpu.make_async_copy(h2_hbm.at[gidx.at[b]],
                                              gbuf.at[b], gsem[b]).wait()
                        pltpu.async_copy(gbuf.at[b], acc_sh.at[ridx2d.at[j]],
                                         ssem[b], add=True)
                        pltpu.make_async_copy(gbuf.at[b],
                                              acc_sh.at[ridx2d.at[j]],
                                              ssem[b]).wait()

                    @pl.when(j + NBUF < nbw)
                    def _(j=j, b=b):
                        pltpu.async_copy(h2_hbm.at[gidx.at[j + NBUF]],
                                         gbuf.at[b], gsem[b])
                return 0
            lax.fori_loop(0, (nblk + NBUF - 1) // NBUF, ring, 0)
            plsc.subcore_barrier()
            pltpu.sync_copy(
                acc_sh.at[my_rows],
                acc_hbm.at[cid, c32 >> 2, my_rows,
                           pl.ds((c32 & 3) * F, F)])
            plsc.subcore_barrier()
            return 0
        lax.fori_loop(0, nch, chunk, 0)

    fn = pl.kernel(
        body,
        out_type=tuple(out_type),
        mesh=mesh,
        scratch_types=(
            pltpu.VMEM_SHARED((rnp, F), jnp.float32),
            pltpu.VMEM((nblk, GB), jnp.int32),
            pltpu.VMEM((nblk, GB), jnp.int32),
            pltpu.VMEM((nblk, GB), jnp.int32),
            pltpu.VMEM((NBUF, GB, F), jnp.float32),
            pltpu.VMEM((GB, F), jnp.float32),
        ) + (pltpu.SemaphoreType.DMA,) * (2 * NBUF),
        compiler_params=pltpu.CompilerParams(use_tc_tiling_on_sc=False),
    )
    return fn(h2, src_p, ridx_p)


def _concat_kernel(x, node_type3, type_emb, *, bn):
    n, d = x.shape
    nt, td = type_emb.shape
    nb = n // bn
    ncg = (d + td) // 128

    def body(x_ref, nt_ref, te_ref, o_ref):
        ids = nt_ref[0]
        temb = jnp.zeros((bn, td), jnp.float32)
        for t in range(nt):
            temb = jnp.where(ids == t, te_ref[t][None, :], temb)
        for g in range(d // 128):
            o_ref[g] = x_ref[:, g * 128:(g + 1) * 128]
        o_ref[ncg - 1] = temb

    return pl.pallas_call(
        body,
        grid=(nb,),
        in_specs=[
            pl.BlockSpec((bn, d), lambda i: (i, 0)),
            pl.BlockSpec((1, bn, 1), lambda i: (i, 0, 0)),
            pl.BlockSpec((nt, td), lambda i: (0, 0)),
        ],
        out_specs=pl.BlockSpec((ncg, bn, 128), lambda i: (0, i, 0)),
        out_shape=jax.ShapeDtypeStruct((ncg, n, 128), jnp.float32),
    )(x, node_type3, type_emb)


def _repack_kernel(acc, deg, *, n_nodes, r_rel, bn):
    _, ncg, rnp, f = acc.shape
    rn = r_rel * n_nodes
    nb = rn // bn

    def body(a_ref, d_ref, o_ref):
        d = d_ref[0, :, 0:1] + d_ref[1, :, 0:1]
        recip = 1.0 / jnp.maximum(d, 1.0)
        o_ref[...] = (a_ref[0, 0] + a_ref[1, 0]) * recip

    return pl.pallas_call(
        body,
        grid=(ncg, nb),
        in_specs=[
            pl.BlockSpec((2, 1, bn, f), lambda c, i: (0, c, i, 0)),
            pl.BlockSpec((2, bn, f), lambda c, i: (0, i, 0)),
        ],
        out_specs=pl.BlockSpec((bn, f), lambda c, i: (i, c)),
        out_shape=jax.ShapeDtypeStruct((rn, ncg * f), jnp.float32),
    )(acc, deg)


def _layer_matmul(h3, aggs, w_self, w_rel, b, *, bn, out_cgm):
    ncg, n, _ = h3.shape
    d_in = ncg * 128
    r_rel = w_rel.shape[0]
    d_out = w_self.shape[1]
    nb = n // bn
    ocg = d_out // 128

    def body(h_ref, a_ref, ws_ref, wr_ref, b_ref, o_ref):
        h_blk = jnp.concatenate([h_ref[g] for g in range(ncg)], axis=1)
        out = jnp.dot(h_blk, ws_ref[...],
                      preferred_element_type=jnp.float32) + b_ref[0][None, :]
        for r in range(r_rel):
            out += jnp.dot(a_ref[r], wr_ref[r],
                           preferred_element_type=jnp.float32)
        out = jnp.maximum(out, 0.0)
        if out_cgm:
            for g in range(ocg):
                o_ref[g] = out[:, g * 128:(g + 1) * 128]
        else:
            o_ref[...] = out

    if out_cgm:
        out_spec = pl.BlockSpec((ocg, bn, 128), lambda i: (0, i, 0))
        out_shape = jax.ShapeDtypeStruct((ocg, n, 128), jnp.float32)
    else:
        out_spec = pl.BlockSpec((bn, d_out), lambda i: (i, 0))
        out_shape = jax.ShapeDtypeStruct((n, d_out), jnp.float32)

    return pl.pallas_call(
        body,
        grid=(nb,),
        in_specs=[
            pl.BlockSpec((ncg, bn, 128), lambda i: (0, i, 0)),
            pl.BlockSpec((r_rel, bn, d_in), lambda i: (0, i, 0)),
            pl.BlockSpec((d_in, d_out), lambda i: (0, 0)),
            pl.BlockSpec((r_rel, d_in, d_out), lambda i: (0, 0, 0)),
            pl.BlockSpec((1, d_out), lambda i: (0, 0)),
        ],
        out_specs=out_spec,
        out_shape=out_shape,
    )(h3, aggs, w_self, w_rel, b)


def kernel(x, edge_index, edge_type, node_type, type_emb,
           W_self0, W_rel0, b0, W_self1, W_rel1, b1):
    n, d = x.shape
    e = edge_index.shape[1]
    r_rel = W_rel0.shape[0]
    d_in0 = d + type_emb.shape[1]

    tb = -(-e // (GB * NWORK)) * NWORK
    pad = tb * GB - e
    nbt = tb // NSUB
    nba = max(1, min(nbt - 1, round(nbt * CORE0_SHARE)))
    nbb = nbt - nba
    nbmax = max(nba, nbb)
    src = jnp.concatenate([edge_index[0], jnp.zeros((pad,), jnp.int32)])
    ridx = edge_type * n + edge_index[1]
    ridx = jnp.concatenate([ridx, jnp.full((pad,), r_rel * n, jnp.int32)])

    def part(a):
        a = a.reshape(tb, GB)
        blk_a = a[:NSUB * nba].reshape(NSUB, nba, GB)
        blk_b = a[NSUB * nba:].reshape(NSUB, nbb, GB)
        blk_a = jnp.pad(blk_a, ((0, 0), (0, nbmax - nba), (0, 0)))
        blk_b = jnp.pad(blk_b, ((0, 0), (0, nbmax - nbb), (0, 0)))
        return jnp.stack([blk_a, blk_b], axis=1).reshape(NWORK, nbmax, GB)
    src_p = part(src)
    ridx_p = part(ridx)
    node_type3 = node_type.reshape(n // 400, 400, 1)

    h0 = _concat_kernel(x, node_type3, type_emb, bn=400)
    ncg0 = d_in0 // 128
    acc0, deg = _sc_aggregate(h0.reshape(ncg0 * n * 4, F), src_p, ridx_p,
                              nba=nba, nbb=nbb, n_nodes=n, ncg=ncg0,
                              r_rel=r_rel, with_deg=True)
    agg0 = _repack_kernel(acc0, deg, n_nodes=n, r_rel=r_rel, bn=2000)
    h1 = _layer_matmul(h0, agg0.reshape(r_rel, n, d_in0),
                       W_self0, W_rel0, b0.reshape(1, -1), bn=400,
                       out_cgm=True)

    ncg1 = h1.shape[0]
    (acc1,) = _sc_aggregate(h1.reshape(ncg1 * n * 4, F), src_p, ridx_p,
                            nba=nba, nbb=nbb, n_nodes=n, ncg=ncg1,
                            r_rel=r_rel, with_deg=False)
    agg1 = _repack_kernel(acc1, deg, n_nodes=n, r_rel=r_rel, bn=2000)
    return _layer_matmul(h1, agg1.reshape(r_rel, n, ncg1 * 128),
                         W_self1, W_rel1, b1.reshape(1, -1), bn=400,
                         out_cgm=False)
